# Initial kernel scaffold; baseline (speedup 1.0000x reference)
import functools

import jax
import jax.numpy as jnp
from jax import lax
from jax.experimental import pallas as pl
from jax.experimental.pallas import tpu as pltpu
from jax.experimental.pallas import tpu_sc as plsc

N_ROWS = 16384
N_COLS = 16384
NNZ = 2684354
D = 64

NC = 2
NS = 16
NW = NC * NS
K = 128
CHUNKS_PER_TILE = 656
M = NW * CHUNKS_PER_TILE
NNZ_PAD = M * K

_mesh = plsc.VectorSubcoreMesh(core_axis_name="c", subcore_axis_name="s")


@functools.partial(
    pl.kernel,
    out_type=jax.ShapeDtypeStruct((NW, N_ROWS // NS, D), jnp.float32),
    mesh=_mesh,
    scratch_types=[
        pltpu.VMEM((1, K), jnp.int32),
        pltpu.VMEM((1, K), jnp.int32),
        pltpu.VMEM((1, K), jnp.float32),
        pltpu.VMEM((K, D), jnp.float32),
        pltpu.VMEM_SHARED((N_ROWS, D), jnp.float32),
        pltpu.SemaphoreType.DMA,
    ],
)
def _sc_spmm(table, gidx, sidx, vals, zeros, out,
             gidx_v, sidx_v, vals_v, rows_v, acc, sem):
    cid = lax.axis_index("c")
    sid = lax.axis_index("s")
    wid = cid * NS + sid
    rpt = N_ROWS // NS

    pltpu.sync_copy(zeros.at[pl.ds(sid * rpt, rpt)],
                    acc.at[pl.ds(sid * rpt, rpt)])
    plsc.subcore_barrier()

    def body(i, carry):
        c = wid * CHUNKS_PER_TILE + i
        pltpu.sync_copy(gidx.at[pl.ds(c, 1)], gidx_v)
        pltpu.sync_copy(sidx.at[pl.ds(c, 1)], sidx_v)
        pltpu.sync_copy(vals.at[pl.ds(c, 1)], vals_v)
        pltpu.async_copy(table.at[gidx_v.at[0]], rows_v, sem).wait()
        zero16 = jnp.zeros((16,), jnp.int32)
        for r in range(K):
            b = plsc.load_gather(vals_v, [zero16, jnp.full((16,), r, jnp.int32)])
            for j in range(D // 16):
                sl = pl.ds(j * 16, 16)
                rows_v[r, sl] = rows_v[r, sl] * b
        pltpu.sync_copy(rows_v, acc.at[sidx_v.at[0]], add=True)
        return carry

    lax.fori_loop(0, CHUNKS_PER_TILE, body, 0)
    plsc.subcore_barrier()
    pltpu.sync_copy(acc.at[pl.ds(sid * rpt, rpt)], out.at[wid])


_BLK = 1024


def _div_body(sino_ref, pp_ref, out_ref):
    out_ref[...] = sino_ref[...] / (pp_ref[0] + pp_ref[1] + 1e-8)


def _temp_proj(sinogram, pp):
    return pl.pallas_call(
        _div_body,
        grid=(N_ROWS // _BLK,),
        in_specs=[
            pl.BlockSpec((_BLK, D), lambda i: (i, 0)),
            pl.BlockSpec((2, _BLK, D), lambda i: (0, i, 0)),
        ],
        out_specs=pl.BlockSpec((_BLK, D), lambda i: (i, 0)),
        out_shape=jax.ShapeDtypeStruct((N_ROWS, D), jnp.float32),
    )(sinogram, pp)


def _final_body(img_ref, eff_ref, pp_ref, out_ref):
    out_ref[...] = img_ref[...] / eff_ref[...] * (pp_ref[0] + pp_ref[1])


def _final(image, eff, pp):
    return pl.pallas_call(
        _final_body,
        grid=(N_COLS // _BLK,),
        in_specs=[
            pl.BlockSpec((_BLK, D), lambda i: (i, 0)),
            pl.BlockSpec((_BLK, D), lambda i: (i, 0)),
            pl.BlockSpec((2, _BLK, D), lambda i: (0, i, 0)),
        ],
        out_specs=pl.BlockSpec((_BLK, D), lambda i: (i, 0)),
        out_shape=jax.ShapeDtypeStruct((N_COLS, D), jnp.float32),
    )(image, eff, pp)


def kernel(image, efficiency_map, sinogram, matrix_vals, matrix_rows, matrix_cols):
    pad = NNZ_PAD - NNZ
    cols2 = jnp.concatenate([matrix_cols, jnp.zeros((pad,), jnp.int32)]).reshape(M, K)
    rows2 = jnp.concatenate([matrix_rows, jnp.zeros((pad,), jnp.int32)]).reshape(M, K)
    vals2 = jnp.concatenate([matrix_vals, jnp.zeros((pad,), jnp.float32)]).reshape(M, K)
    zeros = jnp.zeros((N_ROWS, D), jnp.float32)

    pp = _sc_spmm(image, cols2, rows2, vals2, zeros).reshape(NC, N_ROWS, D)
    temp = _temp_proj(sinogram, pp)
    bp = _sc_spmm(temp, rows2, cols2, vals2, zeros).reshape(NC, N_COLS, D)
    return _final(image, efficiency_map, bp)


# speedup vs baseline: 8.6194x; 8.6194x over previous
import functools

import jax
import jax.numpy as jnp
from jax import lax
from jax.experimental import pallas as pl
from jax.experimental.pallas import tpu as pltpu
from jax.experimental.pallas import tpu_sc as plsc

N_ROWS = 16384
N_COLS = 16384
NNZ = 2684354
D = 64

NC = 2
NS = 16
NW = NC * NS
K = 128
CHUNKS_PER_TILE = 656
M = NW * CHUNKS_PER_TILE
NNZ_PAD = M * K

_mesh = plsc.VectorSubcoreMesh(core_axis_name="c", subcore_axis_name="s")


@functools.partial(
    pl.kernel,
    out_type=jax.ShapeDtypeStruct((NW, N_ROWS // NS, D), jnp.float32),
    mesh=_mesh,
    compiler_params=pltpu.CompilerParams(
        needs_layout_passes=False, use_tc_tiling_on_sc=False),
    scratch_types=[
        pltpu.VMEM((1, K), jnp.int32),
        pltpu.VMEM((1, K), jnp.int32),
        pltpu.VMEM((1, K), jnp.float32),
        pltpu.VMEM((K, D), jnp.float32),
        pltpu.VMEM_SHARED((N_ROWS, D), jnp.float32),
        pltpu.SemaphoreType.DMA,
    ],
)
def _sc_spmm(table, gidx, sidx, vals, zeros, out,
             gidx_v, sidx_v, vals_v, rows_v, acc, sem):
    cid = lax.axis_index("c")
    sid = lax.axis_index("s")
    wid = cid * NS + sid
    rpt = N_ROWS // NS

    pltpu.sync_copy(zeros.at[pl.ds(sid * rpt, rpt)],
                    acc.at[pl.ds(sid * rpt, rpt)])
    plsc.subcore_barrier()

    def body(i, carry):
        c = wid * CHUNKS_PER_TILE + i
        pltpu.sync_copy(gidx.at[pl.ds(c, 1)], gidx_v)
        pltpu.sync_copy(sidx.at[pl.ds(c, 1)], sidx_v)
        pltpu.sync_copy(vals.at[pl.ds(c, 1)], vals_v)
        pltpu.async_copy(table.at[gidx_v.at[0]], rows_v, sem).wait()
        zero16 = jnp.zeros((16,), jnp.int32)
        for r in range(K):
            b = plsc.load_gather(vals_v, [zero16, jnp.full((16,), r, jnp.int32)])
            for j in range(D // 16):
                sl = pl.ds(j * 16, 16)
                rows_v[r, sl] = rows_v[r, sl] * b
        pltpu.sync_copy(rows_v, acc.at[sidx_v.at[0]], add=True)
        return carry

    lax.fori_loop(0, CHUNKS_PER_TILE, body, 0)
    plsc.subcore_barrier()
    pltpu.sync_copy(acc.at[pl.ds(sid * rpt, rpt)], out.at[wid])


_BLK = 1024


def _div_body(sino_ref, pp_ref, out_ref):
    out_ref[...] = sino_ref[...] / (pp_ref[0] + pp_ref[1] + 1e-8)


def _temp_proj(sinogram, pp):
    return pl.pallas_call(
        _div_body,
        grid=(N_ROWS // _BLK,),
        in_specs=[
            pl.BlockSpec((_BLK, D), lambda i: (i, 0)),
            pl.BlockSpec((2, _BLK, D), lambda i: (0, i, 0)),
        ],
        out_specs=pl.BlockSpec((_BLK, D), lambda i: (i, 0)),
        out_shape=jax.ShapeDtypeStruct((N_ROWS, D), jnp.float32),
    )(sinogram, pp)


def _final_body(img_ref, eff_ref, pp_ref, out_ref):
    out_ref[...] = img_ref[...] / eff_ref[...] * (pp_ref[0] + pp_ref[1])


def _final(image, eff, pp):
    return pl.pallas_call(
        _final_body,
        grid=(N_COLS // _BLK,),
        in_specs=[
            pl.BlockSpec((_BLK, D), lambda i: (i, 0)),
            pl.BlockSpec((_BLK, D), lambda i: (i, 0)),
            pl.BlockSpec((2, _BLK, D), lambda i: (0, i, 0)),
        ],
        out_specs=pl.BlockSpec((_BLK, D), lambda i: (i, 0)),
        out_shape=jax.ShapeDtypeStruct((N_COLS, D), jnp.float32),
    )(image, eff, pp)


def kernel(image, efficiency_map, sinogram, matrix_vals, matrix_rows, matrix_cols):
    pad = NNZ_PAD - NNZ
    cols2 = jnp.concatenate([matrix_cols, jnp.zeros((pad,), jnp.int32)]).reshape(M, K)
    rows2 = jnp.concatenate([matrix_rows, jnp.zeros((pad,), jnp.int32)]).reshape(M, K)
    vals2 = jnp.concatenate([matrix_vals, jnp.zeros((pad,), jnp.float32)]).reshape(M, K)
    zeros = jnp.zeros((N_ROWS, D), jnp.float32)

    pp = _sc_spmm(image, cols2, rows2, vals2, zeros).reshape(NC, N_ROWS, D)
    temp = _temp_proj(sinogram, pp)
    bp = _sc_spmm(temp, rows2, cols2, vals2, zeros).reshape(NC, N_COLS, D)
    return _final(image, efficiency_map, bp)
